# speedup vs baseline: 2.3849x; 2.3849x over previous
import functools

import jax
import jax.numpy as jnp
from jax import lax
from jax.experimental import pallas as pl
from jax.experimental.pallas import tpu as pltpu
from jax.experimental.pallas import tpu_sc as plsc

N = 10000
E = 320000
D = 128
DE = 16

NC, NS = 2, 16
NW = NC * NS
EW = E // NW
CH = 80
NIT = EW // CH


def _proj_body(x_ref, wst_ref, wdt_ref, ps_ref, pd_ref):
    xb = x_ref[...]
    ps_ref[...] = jnp.dot(xb, wst_ref[...], preferred_element_type=jnp.float32)
    pd_ref[...] = jnp.dot(xb, wdt_ref[...], preferred_element_type=jnp.float32)


_node_proj = pl.pallas_call(
    _proj_body,
    grid=(10,),
    in_specs=[
        pl.BlockSpec((N // 10, D), lambda i: (i, 0)),
        pl.BlockSpec((D, D), lambda i: (0, 0)),
        pl.BlockSpec((D, D), lambda i: (0, 0)),
    ],
    out_specs=[
        pl.BlockSpec((N // 10, D), lambda i: (i, 0)),
        pl.BlockSpec((N // 10, D), lambda i: (i, 0)),
    ],
    out_shape=[
        jax.ShapeDtypeStruct((N, D), jnp.float32),
        jax.ShapeDtypeStruct((N, D), jnp.float32),
    ],
)


_EB = 3200


def _base_body(e_ref, wet_ref, b_ref, o_ref):
    o_ref[...] = (
        jnp.dot(e_ref[...], wet_ref[...], preferred_element_type=jnp.float32)
        + b_ref[...]
    )


_edge_base = pl.pallas_call(
    _base_body,
    grid=(E // _EB,),
    in_specs=[
        pl.BlockSpec((_EB, DE), lambda i: (i, 0)),
        pl.BlockSpec((DE, D), lambda i: (0, 0)),
        pl.BlockSpec((1, D), lambda i: (0, 0)),
    ],
    out_specs=pl.BlockSpec((_EB, D), lambda i: (i, 0)),
    out_shape=jax.ShapeDtypeStruct((E, D), jnp.float32),
)


@functools.partial(
    pl.kernel,
    out_type=jax.ShapeDtypeStruct((E, D), jnp.float32),
    mesh=plsc.VectorSubcoreMesh(core_axis_name="c", subcore_axis_name="s"),
    scratch_types=[
        pltpu.VMEM((CH,), jnp.int32),
        pltpu.VMEM((CH,), jnp.int32),
        pltpu.VMEM((CH, D), jnp.float32),
        pltpu.VMEM((CH, D), jnp.float32),
        pltpu.VMEM((CH, D), jnp.float32),
        pltpu.SemaphoreType.DMA,
        pltpu.SemaphoreType.DMA,
    ],
)
def _sc_combine(ps_hbm, pd_hbm, src_hbm, dst_hbm, base_hbm, out_hbm,
                idxs_v, idxd_v, ps_v, pd_v, o_v, sem1, sem2):
    wid = lax.axis_index("s") * NC + lax.axis_index("c")
    wbase = wid * EW

    @pl.loop(0, NIT)
    def _chunk(it):
        estart = wbase + it * CH
        pltpu.sync_copy(src_hbm.at[pl.ds(estart, CH)], idxs_v)
        pltpu.sync_copy(dst_hbm.at[pl.ds(estart, CH)], idxd_v)
        g1 = pltpu.async_copy(ps_hbm.at[idxs_v], ps_v, sem1)
        g2 = pltpu.async_copy(pd_hbm.at[idxd_v], pd_v, sem2)
        pltpu.sync_copy(base_hbm.at[pl.ds(estart, CH)], o_v)
        g1.wait()
        g2.wait()

        @pl.loop(0, CH)
        def _row(r):
            for j in range(D // 16):
                sl = pl.ds(j * 16, 16)
                o_v[r, sl] = o_v[r, sl] + ps_v[r, sl] + pd_v[r, sl]

        pltpu.sync_copy(o_v, out_hbm.at[pl.ds(estart, CH)])


def kernel(x, e, edge_index, W, b):
    wet = W[:, :DE].T
    wst = W[:, DE:DE + D].T
    wdt = W[:, DE + D:].T
    src = edge_index[0]
    dst = edge_index[1]
    ps, pd = _node_proj(x, wst, wdt)
    base = _edge_base(e, wet, b.reshape(1, D))
    return _sc_combine(ps, pd, src, dst, base)


# speedup vs baseline: 3.8756x; 1.6251x over previous
import functools

import jax
import jax.numpy as jnp
from jax import lax
from jax.experimental import pallas as pl
from jax.experimental.pallas import tpu as pltpu
from jax.experimental.pallas import tpu_sc as plsc

N = 10000
E = 320000
D = 128
DE = 16

NC, NS = 2, 16
NW = NC * NS
EW = E // NW
CH = 80
NIT = EW // CH


def _proj_body(x_ref, wst_ref, wdt_ref, ps_ref, pd_ref):
    xb = x_ref[...]
    ps_ref[...] = jnp.dot(xb, wst_ref[...], preferred_element_type=jnp.float32)
    pd_ref[...] = jnp.dot(xb, wdt_ref[...], preferred_element_type=jnp.float32)


_node_proj = pl.pallas_call(
    _proj_body,
    grid=(10,),
    in_specs=[
        pl.BlockSpec((N // 10, D), lambda i: (i, 0)),
        pl.BlockSpec((D, D), lambda i: (0, 0)),
        pl.BlockSpec((D, D), lambda i: (0, 0)),
    ],
    out_specs=[
        pl.BlockSpec((N // 10, D), lambda i: (i, 0)),
        pl.BlockSpec((N // 10, D), lambda i: (i, 0)),
    ],
    out_shape=[
        jax.ShapeDtypeStruct((N, D), jnp.float32),
        jax.ShapeDtypeStruct((N, D), jnp.float32),
    ],
)


_EB = 3200


def _base_body(e_ref, wet_ref, b_ref, o_ref):
    o_ref[...] = (
        jnp.dot(e_ref[...], wet_ref[...], preferred_element_type=jnp.float32)
        + b_ref[...]
    )


_edge_base = pl.pallas_call(
    _base_body,
    grid=(E // _EB,),
    in_specs=[
        pl.BlockSpec((_EB, DE), lambda i: (i, 0)),
        pl.BlockSpec((DE, D), lambda i: (0, 0)),
        pl.BlockSpec((1, D), lambda i: (0, 0)),
    ],
    out_specs=pl.BlockSpec((_EB, D), lambda i: (i, 0)),
    out_shape=jax.ShapeDtypeStruct((E, D), jnp.float32),
)


@functools.partial(
    pl.kernel,
    out_type=jax.ShapeDtypeStruct((E, D), jnp.float32),
    mesh=plsc.VectorSubcoreMesh(core_axis_name="c", subcore_axis_name="s"),
    scratch_types=[
        pltpu.VMEM((EW,), jnp.int32),
        pltpu.VMEM((EW,), jnp.int32),
        [pltpu.VMEM((CH, D), jnp.float32)] * 2,
        [pltpu.VMEM((CH, D), jnp.float32)] * 2,
        [pltpu.VMEM((CH, D), jnp.float32)] * 2,
        [pltpu.SemaphoreType.DMA] * 2,
        [pltpu.SemaphoreType.DMA] * 2,
    ],
)
def _sc_combine(ps_hbm, pd_hbm, src_hbm, dst_hbm, base_hbm, out_hbm,
                idxs_v, idxd_v, ps_v, pd_v, o_v, gsem, osem):
    wid = lax.axis_index("s") * NC + lax.axis_index("c")
    wbase = wid * EW

    pltpu.sync_copy(src_hbm.at[pl.ds(wbase, EW)], idxs_v)
    pltpu.sync_copy(dst_hbm.at[pl.ds(wbase, EW)], idxd_v)

    def _issue(it, p, drain_out):
        off = it * CH
        pltpu.async_copy(ps_hbm.at[idxs_v.at[pl.ds(off, CH)]], ps_v[p], gsem[p])
        pltpu.async_copy(pd_hbm.at[idxd_v.at[pl.ds(off, CH)]], pd_v[p], gsem[p])
        if drain_out:
            pltpu.make_async_copy(o_v[p], out_hbm.at[pl.ds(wbase, CH)],
                                  osem[p]).wait()
        pltpu.async_copy(base_hbm.at[pl.ds(wbase + off, CH)], o_v[p], gsem[p])

    def _finish(it, p):
        pltpu.make_async_copy(base_hbm.at[pl.ds(wbase, CH)], ps_v[p],
                              gsem[p]).wait()
        pltpu.make_async_copy(base_hbm.at[pl.ds(wbase, CH)], pd_v[p],
                              gsem[p]).wait()
        pltpu.make_async_copy(base_hbm.at[pl.ds(wbase, CH)], o_v[p],
                              gsem[p]).wait()

        @pl.loop(0, CH)
        def _row(r):
            for j in range(D // 16):
                sl = pl.ds(j * 16, 16)
                o_v[p][r, sl] = o_v[p][r, sl] + ps_v[p][r, sl] + pd_v[p][r, sl]

        pltpu.async_copy(o_v[p], out_hbm.at[pl.ds(wbase + it * CH, CH)],
                         osem[p])

    _issue(0, 0, False)
    _issue(1, 1, False)

    @pl.loop(0, NIT - 3, step=2)
    def _pair(it):
        _finish(it, 0)
        _issue(it + 2, 0, True)
        _finish(it + 1, 1)
        _issue(it + 3, 1, True)

    _finish(NIT - 3, 0)
    _issue(NIT - 1, 0, True)
    _finish(NIT - 2, 1)
    _finish(NIT - 1, 0)
    pltpu.make_async_copy(o_v[0], out_hbm.at[pl.ds(wbase, CH)], osem[0]).wait()
    pltpu.make_async_copy(o_v[1], out_hbm.at[pl.ds(wbase, CH)], osem[1]).wait()


def kernel(x, e, edge_index, W, b):
    wet = W[:, :DE].T
    wst = W[:, DE:DE + D].T
    wdt = W[:, DE + D:].T
    src = edge_index[0]
    dst = edge_index[1]
    ps, pd = _node_proj(x, wst, wdt)
    base = _edge_base(e, wet, b.reshape(1, D))
    return _sc_combine(ps, pd, src, dst, base)


# speedup vs baseline: 4.3282x; 1.1168x over previous
import functools

import jax
import jax.numpy as jnp
from jax import lax
from jax.experimental import pallas as pl
from jax.experimental.pallas import tpu as pltpu
from jax.experimental.pallas import tpu_sc as plsc

N = 10000
E = 320000
D = 128
DE = 16

NC, NS = 2, 16
NW = NC * NS
EW = E // NW
CH = 80
NIT = EW // CH


def _proj_body(x_ref, wst_ref, wdt_ref, ps_ref, pd_ref):
    xb = x_ref[...]
    ps_ref[...] = jnp.dot(xb, wst_ref[...], preferred_element_type=jnp.float32)
    pd_ref[...] = jnp.dot(xb, wdt_ref[...], preferred_element_type=jnp.float32)


_node_proj = pl.pallas_call(
    _proj_body,
    grid=(10,),
    in_specs=[
        pl.BlockSpec((N // 10, D), lambda i: (i, 0)),
        pl.BlockSpec((D, D), lambda i: (0, 0)),
        pl.BlockSpec((D, D), lambda i: (0, 0)),
    ],
    out_specs=[
        pl.BlockSpec((N // 10, D), lambda i: (i, 0)),
        pl.BlockSpec((N // 10, D), lambda i: (i, 0)),
    ],
    out_shape=[
        jax.ShapeDtypeStruct((N, D), jnp.float32),
        jax.ShapeDtypeStruct((N, D), jnp.float32),
    ],
)


@functools.partial(
    pl.kernel,
    out_type=jax.ShapeDtypeStruct((E, D), jnp.float32),
    mesh=plsc.VectorSubcoreMesh(core_axis_name="c", subcore_axis_name="s"),
    scratch_types=[
        pltpu.VMEM((EW,), jnp.int32),
        pltpu.VMEM((EW,), jnp.int32),
        [pltpu.VMEM((CH, D), jnp.float32)] * 2,
        [pltpu.VMEM((CH, D), jnp.float32)] * 2,
        [pltpu.VMEM((CH, D), jnp.float32)] * 2,
        [pltpu.SemaphoreType.DMA] * 2,
        [pltpu.SemaphoreType.DMA] * 2,
    ],
)
def _sc_gather_add(ps_hbm, pd_hbm, src_hbm, dst_hbm, out_hbm,
                   idxs_v, idxd_v, ps_v, pd_v, o_v, gsem, osem):
    wid = lax.axis_index("s") * NC + lax.axis_index("c")
    wbase = wid * EW

    pltpu.sync_copy(src_hbm.at[pl.ds(wbase, EW)], idxs_v)
    pltpu.sync_copy(dst_hbm.at[pl.ds(wbase, EW)], idxd_v)

    def _issue(it, p):
        off = it * CH
        pltpu.async_copy(ps_hbm.at[idxs_v.at[pl.ds(off, CH)]], ps_v[p], gsem[p])
        pltpu.async_copy(pd_hbm.at[idxd_v.at[pl.ds(off, CH)]], pd_v[p], gsem[p])

    def _finish(it, p, drain_out):
        pltpu.make_async_copy(ps_hbm.at[pl.ds(0, CH)], ps_v[p], gsem[p]).wait()
        pltpu.make_async_copy(pd_hbm.at[pl.ds(0, CH)], pd_v[p], gsem[p]).wait()
        if drain_out:
            pltpu.make_async_copy(o_v[p], out_hbm.at[pl.ds(wbase, CH)],
                                  osem[p]).wait()

        @pl.loop(0, CH)
        def _row(r):
            for j in range(D // 16):
                sl = pl.ds(j * 16, 16)
                o_v[p][r, sl] = ps_v[p][r, sl] + pd_v[p][r, sl]

        pltpu.async_copy(o_v[p], out_hbm.at[pl.ds(wbase + it * CH, CH)],
                         osem[p])

    _issue(0, 0)
    _issue(1, 1)
    _finish(0, 0, False)
    _issue(2, 0)
    _finish(1, 1, False)
    _issue(3, 1)

    @pl.loop(2, NIT - 3, step=2)
    def _pair(it):
        _finish(it, 0, True)
        _issue(it + 2, 0)
        _finish(it + 1, 1, True)
        _issue(it + 3, 1)

    _finish(NIT - 3, 0, True)
    _issue(NIT - 1, 0)
    _finish(NIT - 2, 1, True)
    _finish(NIT - 1, 0, True)
    pltpu.make_async_copy(o_v[0], out_hbm.at[pl.ds(wbase, CH)],
                          osem[0]).wait()
    pltpu.make_async_copy(o_v[1], out_hbm.at[pl.ds(wbase, CH)],
                          osem[1]).wait()


_EB = 3200


def _final_body(g_ref, e_ref, wet_ref, b_ref, o_ref):
    o_ref[...] = (
        g_ref[...]
        + jnp.dot(e_ref[...], wet_ref[...], preferred_element_type=jnp.float32)
        + b_ref[...]
    )


_final = pl.pallas_call(
    _final_body,
    grid=(E // _EB,),
    in_specs=[
        pl.BlockSpec((_EB, D), lambda i: (i, 0)),
        pl.BlockSpec((_EB, DE), lambda i: (i, 0)),
        pl.BlockSpec((DE, D), lambda i: (0, 0)),
        pl.BlockSpec((1, D), lambda i: (0, 0)),
    ],
    out_specs=pl.BlockSpec((_EB, D), lambda i: (i, 0)),
    out_shape=jax.ShapeDtypeStruct((E, D), jnp.float32),
)


def kernel(x, e, edge_index, W, b):
    wet = W[:, :DE].T
    wst = W[:, DE:DE + D].T
    wdt = W[:, DE + D:].T
    src = edge_index[0]
    dst = edge_index[1]
    ps, pd = _node_proj(x, wst, wdt)
    g = _sc_gather_add(ps, pd, src, dst)
    return _final(g, e, wet, b.reshape(1, D))
